# Initial kernel scaffold; baseline (speedup 1.0000x reference)
import jax
import jax.numpy as jnp
from jax.experimental import pallas as pl


def kernel(x, coords, weight_params, l_Wqkv, l_bqkv, l_Wo, l_bo, l_g, l_b, g_Wqkv, g_bqkv, g_Wo, g_bo, g_g, g_b):
    raise NotImplementedError("write your pallas kernel here")


# speedup vs baseline: 1.4631x; 1.4631x over previous
import functools
import math

import jax
import jax.numpy as jnp
import numpy as np
from jax import lax
from jax.experimental import pallas as pl
from jax.experimental.pallas import tpu as pltpu
from jax.experimental.pallas import tpu_sc as plsc

HIDDEN = 384
NHEADS = 8
DH = HIDDEN // NHEADS
CLUSTER_SIZE = 64
L = 4096
NBLK = L // 128
SCALE = 1.0 / math.sqrt(DH)
EPS = 1e-5


def _static_plan():
    n_cluster = max(1, L // CLUSTER_SIZE)
    labels = np.random.RandomState(0).randint(0, n_cluster, size=L)
    index = np.argsort(labels, kind="stable")
    window_sizes = np.bincount(labels).tolist()
    new_sizes = []
    for size in window_sizes:
        if size >= CLUSTER_SIZE * 2:
            num_splits = max(1, size // CLUSTER_SIZE)
            q, r = divmod(size, num_splits)
            new_sizes.extend([q + 1 if i < r else q for i in range(num_splits)])
        else:
            new_sizes.append(size)
    sizes = [s for s in new_sizes if s > 0]
    return index.astype(np.int32), sizes


_PERM_NP, _SIZES = _static_plan()
NWIN = len(_SIZES)

_SEG_NP = np.repeat(np.arange(NWIN, dtype=np.int32), _SIZES)

_SEGQ_NP = _SEG_NP.reshape(NBLK, 128, 1)

_SEGK_NP = np.full((NBLK, 1, 3 * 128), -1, dtype=np.int32)
for _i in range(NBLK):
    _lo = (_i - 1) * 128
    _hi = (_i + 2) * 128
    _s = max(_lo, 0)
    _e = min(_hi, L)
    _SEGK_NP[_i, 0, _s - _lo:_e - _lo] = _SEG_NP[_s:_e]

_SMATT_NP = np.zeros((NBLK, NWIN, 128), dtype=np.float32)
for _i in range(NBLK):
    for _r in range(128):
        _w = _SEG_NP[_i * 128 + _r]
        _SMATT_NP[_i, _w, _r] = 1.0 / _SIZES[_w]

_PERM = jnp.asarray(_PERM_NP)
_SEGQ = jnp.asarray(_SEGQ_NP)
_SEGK = jnp.asarray(_SEGK_NP)
_SMATT = jnp.asarray(_SMATT_NP)


def _nt(a, b):
    return lax.dot_general(a, b, (((1,), (1,)), ((), ())),
                           preferred_element_type=jnp.float32)


def _sc_gather(x2d, idx):
    rows_per_w = L // 32
    mesh = plsc.VectorSubcoreMesh(core_axis_name="c", subcore_axis_name="s",
                                  num_cores=2, num_subcores=16)

    @functools.partial(
        pl.kernel,
        out_type=jax.ShapeDtypeStruct((L, HIDDEN), jnp.float32),
        mesh=mesh,
        scratch_types=[
            pltpu.VMEM((rows_per_w,), jnp.int32),
            pltpu.VMEM((rows_per_w, HIDDEN), jnp.float32),
            pltpu.SemaphoreType.DMA,
        ],
    )
    def body(x_hbm, idx_hbm, out_hbm, idx_v, rows_v, sem):
        wid = lax.axis_index("s") * 2 + lax.axis_index("c")
        base = wid * rows_per_w
        pltpu.sync_copy(idx_hbm.at[pl.ds(base, rows_per_w)], idx_v)
        pltpu.async_copy(x_hbm.at[idx_v], rows_v, sem).wait()
        pltpu.sync_copy(rows_v, out_hbm.at[pl.ds(base, rows_per_w)])

    return body(x2d, idx)


def _attend(xq, xkv, wq3, wk3, wv3, bq3, bk3, bv3, wot3, mask):
    o_acc = None
    for h in range(NHEADS):
        qh = _nt(xq, wq3[h]) + bq3[h]
        kh = _nt(xkv, wk3[h]) + bk3[h]
        vh = _nt(xkv, wv3[h]) + bv3[h]
        s = _nt(qh, kh) * SCALE
        if mask is not None:
            s = jnp.where(mask, s, -1e30)
        m = jnp.max(s, axis=1, keepdims=True)
        e = jnp.exp(s - m)
        p = e / jnp.sum(e, axis=1, keepdims=True)
        oh = jnp.dot(p, vh, preferred_element_type=jnp.float32)
        contrib = jnp.dot(oh, wot3[h], preferred_element_type=jnp.float32)
        o_acc = contrib if o_acc is None else o_acc + contrib
    return o_acc


def _layernorm(x, g, b):
    mu = jnp.mean(x, axis=1, keepdims=True)
    xc = x - mu
    var = jnp.mean(xc * xc, axis=1, keepdims=True)
    return xc * lax.rsqrt(var + EPS) * g + b


def _local_body(xs_p, xs_c, xs_n, wq3, wk3, wv3, bq3, bk3, bv3, wot3, bo,
                lg, lb, segq, segk, smatt, refined_ref, r_ref, acc_ref):
    i = pl.program_id(0)
    xq = xs_c[...]
    xkv = jnp.concatenate([xs_p[...], xs_c[...], xs_n[...]], axis=0)
    mask = segq[0] == segk[0]
    o = _attend(xq, xkv, wq3[...], wk3[...], wv3[...], bq3[...], bk3[...],
                bv3[...], wot3[...], mask) + bo[...]
    refined = _layernorm(xq + o, lg[...], lb[...])
    refined_ref[...] = refined
    part = jnp.dot(smatt[0], refined, preferred_element_type=jnp.float32)

    @pl.when(i == 0)
    def _():
        acc_ref[...] = part

    @pl.when(i > 0)
    def _():
        acc_ref[...] = acc_ref[...] + part

    @pl.when(i == NBLK - 1)
    def _():
        r_ref[...] = acc_ref[...]


def _global_body(refined, r, wq3, wk3, wv3, bq3, bk3, bv3, wot3, bo, gg, gb,
                 h_ref, vec_ref):
    i = pl.program_id(0)

    @pl.when(i == 0)
    def _():
        rr = r[...]
        o = _attend(rr, rr, wq3[...], wk3[...], wv3[...], bq3[...], bk3[...],
                    bv3[...], wot3[...], None) + bo[...]
        rp = _layernorm(rr + o, gg[...], gb[...])
        vec_ref[...] = jnp.mean(rp, axis=0, keepdims=True)

    h_ref[...] = refined[...] + vec_ref[...]


def _split_heads(Wqkv, bqkv, Wo):
    wq, wk, wv = jnp.split(Wqkv, 3, axis=0)
    wq3 = wq.reshape(NHEADS, DH, HIDDEN)
    wk3 = wk.reshape(NHEADS, DH, HIDDEN)
    wv3 = wv.reshape(NHEADS, DH, HIDDEN)
    bq, bk, bv = jnp.split(bqkv, 3)
    bq3 = bq.reshape(NHEADS, 1, DH)
    bk3 = bk.reshape(NHEADS, 1, DH)
    bv3 = bv.reshape(NHEADS, 1, DH)
    wot3 = Wo.T.reshape(NHEADS, DH, HIDDEN)
    return wq3, wk3, wv3, bq3, bk3, bv3, wot3


def _tc_pipeline(xs2d, l_Wqkv, l_bqkv, l_Wo, l_bo, l_g, l_b,
                 g_Wqkv, g_bqkv, g_Wo, g_bo, g_g, g_b, interpret=False):
    lwq3, lwk3, lwv3, lbq3, lbk3, lbv3, lwot3 = _split_heads(l_Wqkv, l_bqkv, l_Wo)
    gwq3, gwk3, gwv3, gbq3, gbk3, gbv3, gwot3 = _split_heads(g_Wqkv, g_bqkv, g_Wo)
    lbo = l_bo.reshape(1, HIDDEN)
    lg = l_g.reshape(1, HIDDEN)
    lb = l_b.reshape(1, HIDDEN)
    gbo = g_bo.reshape(1, HIDDEN)
    gg = g_g.reshape(1, HIDDEN)
    gb = g_b.reshape(1, HIDDEN)

    full3 = lambda shp: pl.BlockSpec(shp, lambda i: (0,) * len(shp))
    blk = lambda shp, im: pl.BlockSpec(shp, im)

    refined, r = pl.pallas_call(
        _local_body,
        grid=(NBLK,),
        in_specs=[
            blk((128, HIDDEN), lambda i: (jnp.maximum(i - 1, 0), 0)),
            blk((128, HIDDEN), lambda i: (i, 0)),
            blk((128, HIDDEN), lambda i: (jnp.minimum(i + 1, NBLK - 1), 0)),
            full3((NHEADS, DH, HIDDEN)), full3((NHEADS, DH, HIDDEN)),
            full3((NHEADS, DH, HIDDEN)),
            full3((NHEADS, 1, DH)), full3((NHEADS, 1, DH)),
            full3((NHEADS, 1, DH)),
            full3((NHEADS, DH, HIDDEN)),
            full3((1, HIDDEN)), full3((1, HIDDEN)), full3((1, HIDDEN)),
            blk((1, 128, 1), lambda i: (i, 0, 0)),
            blk((1, 1, 3 * 128), lambda i: (i, 0, 0)),
            blk((1, NWIN, 128), lambda i: (i, 0, 0)),
        ],
        out_specs=[
            blk((128, HIDDEN), lambda i: (i, 0)),
            full3((NWIN, HIDDEN)),
        ],
        out_shape=[
            jax.ShapeDtypeStruct((L, HIDDEN), jnp.float32),
            jax.ShapeDtypeStruct((NWIN, HIDDEN), jnp.float32),
        ],
        scratch_shapes=[pltpu.VMEM((NWIN, HIDDEN), jnp.float32)],
        interpret=interpret,
    )(xs2d, xs2d, xs2d, lwq3, lwk3, lwv3, lbq3, lbk3, lbv3, lwot3, lbo, lg, lb,
      _SEGQ, _SEGK, _SMATT)

    h2d = pl.pallas_call(
        _global_body,
        grid=(NBLK,),
        in_specs=[
            blk((128, HIDDEN), lambda i: (i, 0)),
            full3((NWIN, HIDDEN)),
            full3((NHEADS, DH, HIDDEN)), full3((NHEADS, DH, HIDDEN)),
            full3((NHEADS, DH, HIDDEN)),
            full3((NHEADS, 1, DH)), full3((NHEADS, 1, DH)),
            full3((NHEADS, 1, DH)),
            full3((NHEADS, DH, HIDDEN)),
            full3((1, HIDDEN)), full3((1, HIDDEN)), full3((1, HIDDEN)),
        ],
        out_specs=blk((128, HIDDEN), lambda i: (i, 0)),
        out_shape=jax.ShapeDtypeStruct((L, HIDDEN), jnp.float32),
        scratch_shapes=[pltpu.VMEM((1, HIDDEN), jnp.float32)],
        interpret=interpret,
    )(refined, r, gwq3, gwk3, gwv3, gbq3, gbk3, gbv3, gwot3, gbo, gg, gb)

    return h2d


def kernel(x, coords, weight_params, l_Wqkv, l_bqkv, l_Wo, l_bo, l_g, l_b,
           g_Wqkv, g_bqkv, g_Wo, g_bo, g_g, g_b):
    del coords, weight_params
    x2d = x.reshape(L, HIDDEN)
    xs2d = _sc_gather(x2d, _PERM)
    h2d = _tc_pipeline(xs2d, l_Wqkv, l_bqkv, l_Wo, l_bo, l_g, l_b,
                       g_Wqkv, g_bqkv, g_Wo, g_bo, g_g, g_b)
    return h2d.reshape(1, L, HIDDEN)
